# Initial kernel scaffold; baseline (speedup 1.0000x reference)
import jax
import jax.numpy as jnp
from jax.experimental import pallas as pl


def kernel(spec_ri, norm_w, norm_b, lin_w, lin_b):
    raise NotImplementedError("write your pallas kernel here")


# speedup vs baseline: 1.2009x; 1.2009x over previous
import jax
import jax.numpy as jnp
from jax.experimental import pallas as pl
from jax.experimental.pallas import tpu as pltpu

_BINS = [16] * 20 + [32] * 10 + [64] * 5 + [65]
_NBANDS = len(_BINS)
_E = 128
_EPS = 1e-5
_TB = 512


def _body(xr_ref, xi_ref, wr_ref, wi_ref, bf_ref, ws_ref, o_ref):
    start = 0
    for i, nb in enumerate(_BINS):
        xr_b = xr_ref[0, start:start + nb, :]
        xi_b = xi_ref[0, start:start + nb, :]
        s1 = (jnp.sum(xr_b, axis=0, keepdims=True)
              + jnp.sum(xi_b, axis=0, keepdims=True))
        s2 = (jnp.sum(xr_b * xr_b, axis=0, keepdims=True)
              + jnp.sum(xi_b * xi_b, axis=0, keepdims=True))
        inv_d = 1.0 / (2.0 * nb)
        mu = s1 * inv_d
        var = s2 * inv_d - mu * mu
        rs = jax.lax.rsqrt(var + _EPS)
        wr_b = wr_ref[:, start:start + nb]
        wi_b = wi_ref[:, start:start + nb]
        m = (jnp.dot(wr_b, xr_b, preferred_element_type=jnp.float32)
             + jnp.dot(wi_b, xi_b, preferred_element_type=jnp.float32))
        wcol = ws_ref[:, i:i + 1]
        bcol = bf_ref[:, i:i + 1]
        o_ref[0, :, i, :] = (m - wcol * mu) * rs + bcol
        start += nb


def kernel(spec_ri, norm_w, norm_b, lin_w, lin_b):
    B, n_bins, T, _ = spec_ri.shape
    wr_cols, wi_cols, bf, ws = [], [], [], []
    for i in range(_NBANDS):
        w = lin_w[i] * norm_w[i][None, :]
        bf.append(lin_b[i] + lin_w[i] @ norm_b[i])
        ws.append(jnp.sum(w, axis=1))
        wr_cols.append(w[:, 0::2])
        wi_cols.append(w[:, 1::2])
    wr_cat = jnp.concatenate(wr_cols, axis=1)
    wi_cat = jnp.concatenate(wi_cols, axis=1)
    bf_a = jnp.stack(bf, axis=1)
    ws_a = jnp.stack(ws, axis=1)
    xr = spec_ri[..., 0]
    xi = spec_ri[..., 1]

    nt = T // _TB
    return pl.pallas_call(
        _body,
        grid=(B, nt),
        in_specs=[
            pl.BlockSpec((1, n_bins, _TB), lambda b, t: (b, 0, t)),
            pl.BlockSpec((1, n_bins, _TB), lambda b, t: (b, 0, t)),
            pl.BlockSpec((_E, n_bins), lambda b, t: (0, 0)),
            pl.BlockSpec((_E, n_bins), lambda b, t: (0, 0)),
            pl.BlockSpec((_E, _NBANDS), lambda b, t: (0, 0)),
            pl.BlockSpec((_E, _NBANDS), lambda b, t: (0, 0)),
        ],
        out_specs=pl.BlockSpec((1, _E, _NBANDS, _TB), lambda b, t: (b, 0, 0, t)),
        out_shape=jax.ShapeDtypeStruct((B, _E, _NBANDS, T), jnp.float32),
        compiler_params=pltpu.CompilerParams(
            dimension_semantics=("parallel", "parallel"),
        ),
    )(xr, xi, wr_cat, wi_cat, bf_a, ws_a)


# speedup vs baseline: 1.6361x; 1.3625x over previous
import jax
import jax.numpy as jnp
from jax.experimental import pallas as pl
from jax.experimental.pallas import tpu as pltpu

_BINS = [16] * 20 + [32] * 10 + [64] * 5 + [65]
_NBANDS = len(_BINS)
_E = 128
_EPS = 1e-5
_TB = 512


def _body(x_ref, wr_ref, wi_ref, bf_ref, ws_ref, o_ref):
    xr = x_ref[0, 0]
    xi = x_ref[0, 1]
    start = 0
    for i, nb in enumerate(_BINS):
        xr_b = xr[start:start + nb, :]
        xi_b = xi[start:start + nb, :]
        s1 = (jnp.sum(xr_b, axis=0, keepdims=True)
              + jnp.sum(xi_b, axis=0, keepdims=True))
        s2 = (jnp.sum(xr_b * xr_b, axis=0, keepdims=True)
              + jnp.sum(xi_b * xi_b, axis=0, keepdims=True))
        inv_d = 1.0 / (2.0 * nb)
        mu = s1 * inv_d
        var = s2 * inv_d - mu * mu
        rs = jax.lax.rsqrt(var + _EPS)
        wr_b = wr_ref[:, start:start + nb]
        wi_b = wi_ref[:, start:start + nb]
        m = (jnp.dot(wr_b, xr_b, preferred_element_type=jnp.float32)
             + jnp.dot(wi_b, xi_b, preferred_element_type=jnp.float32))
        wcol = ws_ref[:, i:i + 1]
        bcol = bf_ref[:, i:i + 1]
        o_ref[0, :, i, :] = (m - wcol * mu) * rs + bcol
        start += nb


def kernel(spec_ri, norm_w, norm_b, lin_w, lin_b):
    B, n_bins, T, _ = spec_ri.shape
    wr_cols, wi_cols, bf, ws = [], [], [], []
    for i in range(_NBANDS):
        w = lin_w[i] * norm_w[i][None, :]
        bf.append(lin_b[i] + lin_w[i] @ norm_b[i])
        ws.append(jnp.sum(w, axis=1))
        wr_cols.append(w[:, 0::2])
        wi_cols.append(w[:, 1::2])
    wr_cat = jnp.concatenate(wr_cols, axis=1)
    wi_cat = jnp.concatenate(wi_cols, axis=1)
    bf_a = jnp.stack(bf, axis=1)
    ws_a = jnp.stack(ws, axis=1)
    xri = jnp.transpose(spec_ri, (0, 3, 1, 2))

    nt = T // _TB
    return pl.pallas_call(
        _body,
        grid=(B, nt),
        in_specs=[
            pl.BlockSpec((1, 2, n_bins, _TB), lambda b, t: (b, 0, 0, t)),
            pl.BlockSpec((_E, n_bins), lambda b, t: (0, 0)),
            pl.BlockSpec((_E, n_bins), lambda b, t: (0, 0)),
            pl.BlockSpec((_E, _NBANDS), lambda b, t: (0, 0)),
            pl.BlockSpec((_E, _NBANDS), lambda b, t: (0, 0)),
        ],
        out_specs=pl.BlockSpec((1, _E, _NBANDS, _TB), lambda b, t: (b, 0, 0, t)),
        out_shape=jax.ShapeDtypeStruct((B, _E, _NBANDS, T), jnp.float32),
        compiler_params=pltpu.CompilerParams(
            dimension_semantics=("parallel", "parallel"),
        ),
    )(xri, wr_cat, wi_cat, bf_a, ws_a)
